# baseline (speedup 1.0000x reference)
import functools

import jax
import jax.numpy as jnp
import numpy as np
from jax import lax
from jax.experimental import pallas as pl
from jax.experimental.pallas import tpu as pltpu
from jax.experimental.pallas import tpu_sc as plsc

_GAMMA = np.float32(12.0)
_PI = 3.141592653589793
_EMB_INIT = 0.21875
_INV2 = np.float32(2.0 / _EMB_INIT)
_BIG = np.float32(1.5 * 2 ** 23)
_HI = np.float32(_PI / 2)
_S1 = np.float32(-1.0 / 6)
_S2 = np.float32(1.0 / 120)
_C1 = np.float32(-0.5)
_C2 = np.float32(1.0 / 24)
_ONE = np.float32(1.0)
_HALF = np.float32(0.5)
_THREE_HALVES = np.float32(1.5)
_RSQRT_MAGIC = np.int32(0x5F3759DF)
_BF16_RN = np.int32(0x8000)
_HI_MASK = np.int32(np.uint32(0xFFFF0000).astype(np.int32))

_NC = 2
_NS = 16
_NW = _NC * _NS
_D = 128
_H = 64
_C = 80
_G = 16
_TC_ROWS = 1280
_SPLIT = 156160


def _tc_sincos_body(attr_ref, cs_ref):
    at = attr_ref[...]
    t = at * _INV2
    tb = t + _BIG
    k = lax.bitcast_convert_type(tb, jnp.int32)
    r0 = tb - _BIG
    r = (t - r0) * _HI
    z = r * r
    sinp = r * (_ONE + z * (_S1 + z * _S2))
    cosp = _ONE + z * (_C1 + z * _C2)
    swap = (k & 1) == 1
    sb = jnp.where(swap, cosp, sinp)
    cb = jnp.where(swap, sinp, cosp)
    s = lax.bitcast_convert_type(sb, jnp.int32) ^ ((k & 2) << 30)
    c = lax.bitcast_convert_type(cb, jnp.int32) ^ ((k ^ (k >> 1)) << 31)
    cs_ref[...] = ((c + _BF16_RN) & _HI_MASK) | (
        lax.shift_right_logical(s + _BF16_RN, 16))


def _tc_sincos(attr2):
    n = attr2.shape[0]
    spec = pl.BlockSpec((_TC_ROWS, _D), lambda i: (i, 0))
    return pl.pallas_call(
        _tc_sincos_body,
        grid=(n // _TC_ROWS,),
        in_specs=[spec],
        out_specs=spec,
        out_shape=jax.ShapeDtypeStruct(attr2.shape, jnp.int32),
    )(attr2)


def _sqrt(m2):
    i = _RSQRT_MAGIC - (plsc.bitcast(m2, jnp.int32) >> 1)
    y = plsc.bitcast(i, jnp.float32)
    y = y * (_THREE_HALVES - (_HALF * m2) * y * y)
    return m2 * y


def _rot(v, idx):
    return jnp.take_along_axis(v, idx, axis=0,
                               mode=lax.GatherScatterMode.PROMISE_IN_BOUNDS)


def _slice_term(head_v, tail_v, cs_v, e, f):
    rh = head_v[e, pl.ds(f * _G, _G)]
    ih = head_v[e, pl.ds(_H + f * _G, _G)]
    rt = tail_v[e, pl.ds(f * _G, _G)]
    it = tail_v[e, pl.ds(_H + f * _G, _G)]
    cs = cs_v[e >> 1, pl.ds((e & 1) * _H + f * _G, _G)]
    c = plsc.bitcast(cs & _HI_MASK, jnp.float32)
    s = plsc.bitcast(cs << 16, jnp.float32)
    rs = rh * c - ih * s - rt
    im = rh * s + ih * c - it
    return _sqrt(rs * rs + im * im)


def _issue(ci, wbase2, x_hbm, cs_hbm, src_v, dst_v, head_v, tail_v, cs_buf,
           sem):
    base = ci * _C
    pltpu.async_copy(x_hbm.at[src_v.at[pl.ds(base, _C)]], head_v, sem)
    pltpu.async_copy(x_hbm.at[dst_v.at[pl.ds(base, _C)]], tail_v, sem)
    pltpu.async_copy(cs_hbm.at[pl.ds(wbase2 + ci * (_C // 2), _C // 2)],
                     cs_buf, sem)


def _drain(x_hbm, cs_hbm, src_v, head_v, tail_v, cs_buf, sem):
    idx = src_v.at[pl.ds(0, _C)]
    pltpu.make_async_copy(x_hbm.at[idx], head_v, sem).wait()
    pltpu.make_async_copy(x_hbm.at[idx], tail_v, sem).wait()
    pltpu.make_async_copy(cs_hbm.at[pl.ds(0, _C // 2)], cs_buf, sem).wait()


def _compute_chunk(ci, head_v, tail_v, cs_v, out_v):
    base = ci * _C
    lane = lax.iota(jnp.int32, _G)
    half = lane & 8
    r8 = (lane + 8) & 15
    r4 = half | ((lane + 4) & 7)
    r2 = half | ((lane + 2) & 7)
    r1 = half | ((lane + 1) & 7)
    low_half = lane < 8
    lane7 = lane & 7

    def edge_partial(e):
        acc = (_slice_term(head_v, tail_v, cs_v, e, 0)
               + _slice_term(head_v, tail_v, cs_v, e, 1))
        return acc + (_slice_term(head_v, tail_v, cs_v, e, 2)
                      + _slice_term(head_v, tail_v, cs_v, e, 3))

    def group_body(g, carry):

        def edge_body(i, outacc):
            e0 = g * _G + i
            a = edge_partial(e0)
            b = edge_partial(e0 + 8)
            m = jnp.where(low_half, a + _rot(a, r8), b + _rot(b, r8))
            m = m + _rot(m, r4)
            m = m + _rot(m, r2)
            m = m + _rot(m, r1)
            return jnp.where(lane7 == jnp.full((_G,), i, jnp.int32),
                             _GAMMA - m, outacc)

        outacc = lax.fori_loop(0, _G // 2, edge_body,
                               jnp.zeros((_G,), jnp.float32))
        out_v[pl.ds(base + g * _G, _G)] = outacc
        return carry

    return lax.fori_loop(0, _C // _G, group_body, jnp.int32(0))


def _sc_kernel(e_per_w, x_hbm, src_hbm, dst_hbm, cs_hbm, out_hbm,
               src_v, dst_v, out_v,
               head0, tail0, cs0, head1, tail1, cs1, sem0, sem1):
    wid = lax.axis_index("s") * _NC + lax.axis_index("c")
    wbase = wid * e_per_w
    wbase2 = wid * (e_per_w // 2)
    n_chunks = e_per_w // _C

    pltpu.sync_copy(src_hbm.at[pl.ds(wbase, e_per_w)], src_v)
    pltpu.sync_copy(dst_hbm.at[pl.ds(wbase, e_per_w)], dst_v)

    buf0 = (head0, tail0, cs0, sem0)
    buf1 = (head1, tail1, cs1, sem1)

    def issue(ci, buf):
        h, t, cs, sem = buf
        _issue(ci, wbase2, x_hbm, cs_hbm, src_v, dst_v, h, t, cs, sem)

    def drain_compute(ci, buf):
        h, t, cs, sem = buf
        _drain(x_hbm, cs_hbm, src_v, h, t, cs, sem)
        _compute_chunk(ci, h, t, cs, out_v)

    n_pairs = (n_chunks - 1) // 2
    issue(0, buf0)

    def pair_body(i, carry):
        issue(2 * i + 1, buf1)
        drain_compute(2 * i, buf0)
        issue(2 * i + 2, buf0)
        drain_compute(2 * i + 1, buf1)
        return carry

    lax.fori_loop(0, n_pairs, pair_body, jnp.int32(0))
    if n_chunks % 2 == 1:
        drain_compute(n_chunks - 1, buf0)
    else:
        issue(n_chunks - 1, buf1)
        drain_compute(n_chunks - 2, buf0)
        drain_compute(n_chunks - 1, buf1)

    pltpu.sync_copy(out_v, out_hbm.at[pl.ds(wbase, e_per_w)])


def _sc_score(x, src, dst, cs, n_edges):
    e_per_w = n_edges // _NW
    mesh = plsc.VectorSubcoreMesh(
        core_axis_name="c", subcore_axis_name="s",
        num_cores=_NC, num_subcores=_NS)
    run = pl.kernel(
        functools.partial(_sc_kernel, e_per_w),
        out_type=jax.ShapeDtypeStruct((n_edges,), jnp.float32),
        mesh=mesh,
        compiler_params=pltpu.CompilerParams(needs_layout_passes=False),
        scratch_types=[
            pltpu.VMEM((e_per_w,), jnp.int32),
            pltpu.VMEM((e_per_w,), jnp.int32),
            pltpu.VMEM((e_per_w,), jnp.float32),
            pltpu.VMEM((_C, _D), jnp.float32),
            pltpu.VMEM((_C, _D), jnp.float32),
            pltpu.VMEM((_C // 2, _D), jnp.int32),
            pltpu.VMEM((_C, _D), jnp.float32),
            pltpu.VMEM((_C, _D), jnp.float32),
            pltpu.VMEM((_C // 2, _D), jnp.int32),
            pltpu.SemaphoreType.DMA,
            pltpu.SemaphoreType.DMA,
        ],
    )
    return run(x, src, dst, cs)


def kernel(x, edge_index, edge_attr):
    n_edges = edge_index.shape[1]
    src = edge_index[0].astype(jnp.int32)
    dst = edge_index[1].astype(jnp.int32)
    attr = edge_attr.astype(jnp.float32)

    grain = _NW * _C
    n_parts = 3
    units = n_edges // grain
    sizes = [(units // n_parts) * grain] * n_parts
    sizes[-1] += n_edges - sum(sizes)
    outs, lo = [], 0
    for n_p in sizes:
        hi = lo + n_p
        cs_p = _tc_sincos(attr[lo:hi].reshape(n_p // 2, _D))
        outs.append(_sc_score(x, src[lo:hi], dst[lo:hi], cs_p, n_p))
        lo = hi
    return jnp.concatenate(outs)


# speedup vs baseline: 1.1167x; 1.1167x over previous
import functools

import jax
import jax.numpy as jnp
import numpy as np
from jax import lax
from jax.experimental import pallas as pl
from jax.experimental.pallas import tpu as pltpu
from jax.experimental.pallas import tpu_sc as plsc

_GAMMA = np.float32(12.0)
_PI = 3.141592653589793
_EMB_INIT = 0.21875
_INV2 = np.float32(2.0 / _EMB_INIT)
_BIG = np.float32(1.5 * 2 ** 23)
_HI = np.float32(_PI / 2)
_S1 = np.float32(-1.0 / 6)
_S2 = np.float32(1.0 / 120)
_S1M = np.float32(-0.16605)
_C1 = np.float32(-0.5)
_C2 = np.float32(1.0 / 24)
_HALF = np.float32(0.5)
_ONE = np.float32(1.0)
_THREE_HALVES = np.float32(1.5)
_RSQRT_MAGIC = np.int32(0x5F3759DF)

_NC = 2
_NS = 16
_NW = _NC * _NS
_D = 128
_H = 64
_C = 80
_G = 16
_UNROLL = 8


def _sincos_parts(at, ck_tab, sk_tab):
    t = at * _INV2
    tb = t + _BIG
    k = plsc.bitcast(tb, jnp.int32)
    r0 = tb - _BIG
    r = (t - r0) * _HI
    z = r * r
    sinp = r * (_ONE + z * _S1M)
    cosp = _ONE + z * (_C1 + z * _C2)
    q = k & 3
    return sinp, cosp, _rot(ck_tab, q), _rot(sk_tab, q)


def _sqrt(m2):
    i = _RSQRT_MAGIC - (plsc.bitcast(m2, jnp.int32) >> 1)
    y = plsc.bitcast(i, jnp.float32)
    y = y * (_THREE_HALVES - (_HALF * m2) * y * y)
    return m2 * y


def _slice_term(head_v, tail_v, attr_v, e, f, ck_tab, sk_tab):
    rh = head_v[e, pl.ds(f * _G, _G)]
    ih = head_v[e, pl.ds(_H + f * _G, _G)]
    rt = tail_v[e, pl.ds(f * _G, _G)]
    it = tail_v[e, pl.ds(_H + f * _G, _G)]
    at = attr_v[e, pl.ds(f * _G, _G)]
    sinp, cosp, ck, sk = _sincos_parts(at, ck_tab, sk_tab)
    vr = rh * cosp - ih * sinp
    vi = rh * sinp + ih * cosp
    rs = ck * vr - sk * vi - rt
    im = sk * vr + ck * vi - it
    return _sqrt(rs * rs + im * im)


def _rot(v, idx):
    return jnp.take_along_axis(v, idx, axis=0,
                               mode=lax.GatherScatterMode.PROMISE_IN_BOUNDS)


def _issue(ci, wbase, x_hbm, attr_hbm, src_v, dst_v, head_v, tail_v, attr_v,
           sem):
    base = ci * _C
    pltpu.async_copy(x_hbm.at[src_v.at[pl.ds(base, _C)]], head_v, sem)
    pltpu.async_copy(x_hbm.at[dst_v.at[pl.ds(base, _C)]], tail_v, sem)
    pltpu.async_copy(attr_hbm.at[pl.ds(wbase + base, _C)], attr_v, sem)


def _drain(x_hbm, attr_hbm, src_v, head_v, tail_v, attr_v, sem):
    idx = src_v.at[pl.ds(0, _C)]
    pltpu.make_async_copy(x_hbm.at[idx], head_v, sem).wait()
    pltpu.make_async_copy(x_hbm.at[idx], tail_v, sem).wait()
    pltpu.make_async_copy(attr_hbm.at[pl.ds(0, _C)], attr_v, sem).wait()


def _compute_chunk(ci, head_v, tail_v, attr_v, out_v):
    base = ci * _C
    lane = lax.iota(jnp.int32, _G)
    half = lane & 8
    r8 = (lane + 8) & 15
    r4 = half | ((lane + 4) & 7)
    r2 = half | ((lane + 2) & 7)
    r1 = half | ((lane + 1) & 7)
    low_half = lane < 8
    lane7 = lane & 7
    ck_tab = ((1 - (lane & 2)) * (1 - (lane & 1))).astype(jnp.float32)
    sk_tab = ((lane & 1) * (1 - (lane & 2))).astype(jnp.float32)

    def edge_partial(e):
        acc = (_slice_term(head_v, tail_v, attr_v, e, 0, ck_tab, sk_tab)
               + _slice_term(head_v, tail_v, attr_v, e, 1, ck_tab, sk_tab))
        return acc + (_slice_term(head_v, tail_v, attr_v, e, 2, ck_tab, sk_tab)
                      + _slice_term(head_v, tail_v, attr_v, e, 3, ck_tab, sk_tab))

    def group_body(g, carry):

        def edge_body(i, outacc):
            e0 = g * _G + i
            a = edge_partial(e0)
            b = edge_partial(e0 + 8)
            m = jnp.where(low_half, a + _rot(a, r8), b + _rot(b, r8))
            m = m + _rot(m, r4)
            m = m + _rot(m, r2)
            m = m + _rot(m, r1)
            return jnp.where(lane7 == jnp.full((_G,), i, jnp.int32),
                             _GAMMA - m, outacc)

        outacc = lax.fori_loop(0, _G // 2, edge_body,
                               jnp.zeros((_G,), jnp.float32))
        out_v[pl.ds(base + g * _G, _G)] = outacc
        return carry

    return lax.fori_loop(0, _C // _G, group_body, jnp.int32(0))


def _sc_kernel(e_per_w, x_hbm, src_hbm, dst_hbm, attr_hbm, out_hbm,
               src_v, dst_v, out_v,
               head0, tail0, attr0, head1, tail1, attr1, sem0, sem1):
    wid = lax.axis_index("s") * _NC + lax.axis_index("c")
    wbase = wid * e_per_w
    n_chunks = e_per_w // _C

    pltpu.sync_copy(src_hbm.at[pl.ds(wbase, e_per_w)], src_v)
    pltpu.sync_copy(dst_hbm.at[pl.ds(wbase, e_per_w)], dst_v)

    buf0 = (head0, tail0, attr0, sem0)
    buf1 = (head1, tail1, attr1, sem1)

    def issue(ci, buf):
        h, t, a, sem = buf
        _issue(ci, wbase, x_hbm, attr_hbm, src_v, dst_v, h, t, a, sem)

    def drain_compute(ci, buf):
        h, t, a, sem = buf
        _drain(x_hbm, attr_hbm, src_v, h, t, a, sem)
        _compute_chunk(ci, h, t, a, out_v)

    issue(0, buf0)

    def pair_body(i, carry):
        issue(2 * i + 1, buf1)
        drain_compute(2 * i, buf0)
        issue(2 * i + 2, buf0)
        drain_compute(2 * i + 1, buf1)
        return carry

    lax.fori_loop(0, (n_chunks - 1) // 2, pair_body, jnp.int32(0))
    drain_compute(n_chunks - 1, buf0)

    pltpu.sync_copy(out_v, out_hbm.at[pl.ds(wbase, e_per_w)])


def kernel(x, edge_index, edge_attr):
    n_edges = edge_index.shape[1]
    e_per_w = n_edges // _NW
    src = edge_index[0].astype(jnp.int32)
    dst = edge_index[1].astype(jnp.int32)
    attr = edge_attr.astype(jnp.float32)

    mesh = plsc.VectorSubcoreMesh(
        core_axis_name="c", subcore_axis_name="s",
        num_cores=_NC, num_subcores=_NS)
    run = pl.kernel(
        functools.partial(_sc_kernel, e_per_w),
        out_type=jax.ShapeDtypeStruct((n_edges,), jnp.float32),
        mesh=mesh,
        compiler_params=pltpu.CompilerParams(needs_layout_passes=False),
        scratch_types=[
            pltpu.VMEM((e_per_w,), jnp.int32),
            pltpu.VMEM((e_per_w,), jnp.int32),
            pltpu.VMEM((e_per_w,), jnp.float32),
            pltpu.VMEM((_C, _D), jnp.float32),
            pltpu.VMEM((_C, _D), jnp.float32),
            pltpu.VMEM((_C, _H), jnp.float32),
            pltpu.VMEM((_C, _D), jnp.float32),
            pltpu.VMEM((_C, _D), jnp.float32),
            pltpu.VMEM((_C, _H), jnp.float32),
            pltpu.SemaphoreType.DMA,
            pltpu.SemaphoreType.DMA,
        ],
    )
    return run(x, src, dst, attr)
